# baseline (speedup 1.0000x reference)
import numpy as np

import jax
import jax.numpy as jnp
from jax import lax
from jax.experimental import pallas as pl
from jax.experimental.pallas import tpu as pltpu
from jax.experimental.pallas import tpu_sc as plsc

N = 10000
E = 320000
D = 128
DW = D // 2

NC = 2
NS = 16
NW = NC * NS
EPW = E // NW
CHUNK = 80
NCHUNK = EPW // CHUNK
NSB = 5
SBC = NCHUNK // NSB
ZR = 80
NZB = N // ZR

_PERM = np.concatenate(
    [np.concatenate([32 * j + 2 * np.arange(16),
                     32 * j + 2 * np.arange(16) + 1]) for j in range(4)]
).astype(np.int32)
_PERM2 = _PERM[_PERM]
_PM = np.eye(D, dtype=np.float32)[_PERM].T


def _spmm_body(src_hbm, dst_hbm, vals_hbm, table_hbm, out_hbm,
               acc, srcv, dstv, valv, g0, g1, g2, rowsf,
               gsem0, gsem1, gsem2):
    cid = lax.axis_index("c")
    sid = lax.axis_index("s")
    wid = cid * NS + sid
    gbufs = (g0, g1, g2)
    gsems = (gsem0, gsem1, gsem2)

    zero16 = jnp.zeros((16,), jnp.float32)

    def zb(i, c):
        for j in range(8):
            rowsf[i, pl.ds(j * 16, 16)] = zero16
        return c

    lax.fori_loop(0, ZR, zb, 0)

    for k in range((NZB + NS - 1) // NS):
        b = k * NS + sid

        @pl.when(b < NZB)
        def _():
            base = pl.multiple_of(b * ZR, 8)
            pltpu.sync_copy(rowsf, acc.at[pl.ds(base, ZR)])

    plsc.subcore_barrier()

    def issue_gather(g, buf):
        pltpu.async_copy(table_hbm.at[srcv.at[g]], gbufs[buf], gsems[buf])

    def wait_gather(g, buf):
        pltpu.make_async_copy(
            table_hbm.at[srcv.at[g]], gbufs[buf], gsems[buf]).wait()

    hi_mask = jnp.full((16,), -65536, jnp.int32)

    def scale(g, buf):
        def grp_body(grp, cc):
            vv = valv[g, pl.ds(grp * 16, 16)]
            rbase = grp * 16
            for r in range(16):
                v = vv[r]
                for j in range(4):
                    w = gbufs[buf][rbase + r, pl.ds(j * 16, 16)]
                    lo = lax.bitcast_convert_type(w << 16, jnp.float32)
                    hi = lax.bitcast_convert_type(w & hi_mask, jnp.float32)
                    rowsf[rbase + r, pl.ds(j * 32, 16)] = lo * v
                    rowsf[rbase + r, pl.ds(j * 32 + 16, 16)] = hi * v
            return cc

        lax.fori_loop(0, CHUNK // 16, grp_body, 0)

    def super_block(sb, c):
        pltpu.sync_copy(src_hbm.at[wid, sb], srcv)
        pltpu.sync_copy(dst_hbm.at[wid, sb], dstv)
        pltpu.sync_copy(vals_hbm.at[wid, sb], valv)
        issue_gather(0, 0)
        issue_gather(1, 1)

        def do_chunk(g, buf, issue_ahead):
            wait_gather(g, buf)
            scale(g, buf)
            if issue_ahead:
                issue_gather(g + 2, (buf + 2) % 3)
            pltpu.sync_copy(rowsf, acc.at[dstv.at[g]], add=True)

        def triple(i, cc):
            gb = i * 3
            do_chunk(gb, 0, True)
            do_chunk(gb + 1, 1, True)
            do_chunk(gb + 2, 2, True)
            return cc

        lax.fori_loop(0, (SBC - 4) // 3, triple, 0)
        do_chunk(SBC - 4, (SBC - 4) % 3, True)
        do_chunk(SBC - 3, (SBC - 3) % 3, True)
        do_chunk(SBC - 2, (SBC - 2) % 3, False)
        do_chunk(SBC - 1, (SBC - 1) % 3, False)
        return c

    lax.fori_loop(0, NSB, super_block, 0)
    plsc.subcore_barrier()

    for k in range((NZB + NS - 1) // NS):
        b = k * NS + sid

        @pl.when(b < NZB)
        def _():
            base = pl.multiple_of(b * ZR, 8)
            pltpu.sync_copy(acc.at[pl.ds(base, ZR)],
                            out_hbm.at[cid, pl.ds(base, ZR)])


_spmm = pl.kernel(
    _spmm_body,
    out_type=jax.ShapeDtypeStruct((NC, N, D), jnp.float32),
    mesh=plsc.VectorSubcoreMesh(core_axis_name="c", subcore_axis_name="s"),
    compiler_params=pltpu.CompilerParams(use_tc_tiling_on_sc=False),
    scratch_types=[
        pltpu.VMEM_SHARED((N, D), jnp.float32),
        pltpu.VMEM((SBC, CHUNK), jnp.int32),
        pltpu.VMEM((SBC, CHUNK), jnp.int32),
        pltpu.VMEM((SBC, CHUNK), jnp.float32),
        pltpu.VMEM((CHUNK, DW), jnp.int32),
        pltpu.VMEM((CHUNK, DW), jnp.int32),
        pltpu.VMEM((CHUNK, DW), jnp.int32),
        pltpu.VMEM((CHUNK, D), jnp.float32),
        pltpu.SemaphoreType.DMA,
        pltpu.SemaphoreType.DMA,
        pltpu.SemaphoreType.DMA,
    ],
)


def _pack(x):
    return lax.bitcast_convert_type(
        x.astype(jnp.bfloat16).reshape(N, DW, 2), jnp.int32)


def _tc1_body(lf_ref, f_ref, pm_ref, w1p_ref, b1_ref, inter_ref, part1_ref):
    lf_p = lf_ref[0] + lf_ref[1]
    f_p = lax.dot_general(
        f_ref[...], pm_ref[...], (((1,), (0,)), ((), ())),
        preferred_element_type=jnp.float32)
    inter_ref[...] = lf_p * f_p
    part1_ref[...] = lax.dot_general(
        lf_p + f_p, w1p_ref[...], (((1,), (0,)), ((), ())),
        preferred_element_type=jnp.float32) + b1_ref[...]


BR = 2000

_tc1 = pl.pallas_call(
    _tc1_body,
    grid=(N // BR,),
    in_specs=[
        pl.BlockSpec((NC, BR, D), lambda i: (0, i, 0)),
        pl.BlockSpec((BR, D), lambda i: (i, 0)),
        pl.BlockSpec((D, D), lambda i: (0, 0)),
        pl.BlockSpec((D, D), lambda i: (0, 0)),
        pl.BlockSpec((1, D), lambda i: (0, 0)),
    ],
    out_specs=[
        pl.BlockSpec((BR, D), lambda i: (i, 0)),
        pl.BlockSpec((BR, D), lambda i: (i, 0)),
    ],
    out_shape=[
        jax.ShapeDtypeStruct((N, D), jnp.float32),
        jax.ShapeDtypeStruct((N, D), jnp.float32),
    ],
)


def _tc2_body(part1_ref, p_ref, w2p2_ref, b2_ref, out_ref):
    p = p_ref[0] + p_ref[1]
    out_ref[...] = part1_ref[...] + lax.dot_general(
        p, w2p2_ref[...], (((1,), (0,)), ((), ())),
        preferred_element_type=jnp.float32) + b2_ref[...]


_tc2 = pl.pallas_call(
    _tc2_body,
    grid=(N // BR,),
    in_specs=[
        pl.BlockSpec((BR, D), lambda i: (i, 0)),
        pl.BlockSpec((NC, BR, D), lambda i: (0, i, 0)),
        pl.BlockSpec((D, D), lambda i: (0, 0)),
        pl.BlockSpec((1, D), lambda i: (0, 0)),
    ],
    out_specs=pl.BlockSpec((BR, D), lambda i: (i, 0)),
    out_shape=jax.ShapeDtypeStruct((N, D), jnp.float32),
)


def kernel(laplacian_indices, laplacian_values, features, W1, b1, W2, b2):
    dst = laplacian_indices[0].reshape(NW, NSB, SBC, CHUNK)
    src = laplacian_indices[1].reshape(NW, NSB, SBC, CHUNK)
    vals = laplacian_values.reshape(NW, NSB, SBC, CHUNK)
    pm = jnp.asarray(_PM)
    w1p = W1.T[jnp.asarray(_PERM)]
    w2p2 = W2.T[jnp.asarray(_PERM2)]
    lf_parts = _spmm(src, dst, vals, _pack(features))
    inter_p, part1 = _tc1(lf_parts, features, pm, w1p, b1.reshape(1, D))
    p_parts = _spmm(src, dst, vals, _pack(inter_p))
    return _tc2(part1, p_parts, w2p2, b2.reshape(1, D))


# speedup vs baseline: 2.0266x; 2.0266x over previous
import jax
import jax.numpy as jnp
from jax import lax
from jax.experimental import pallas as pl
from jax.experimental.pallas import tpu as pltpu
from jax.experimental.pallas import tpu_sc as plsc

N = 10000
E = 320000
D = 128

NC = 2
NS = 16
NW = NC * NS
EPW = E // NW
CHUNK = 80
NCHUNK = EPW // CHUNK
NSB = 5
SBC = NCHUNK // NSB
ZR = 80
NZB = N // ZR


def _spmm_body(src_hbm, dst_hbm, vals_hbm, table_hbm, out_hbm,
               acc, srcv, dstv, valv, rows0, rows1, rows2,
               gsem0, gsem1, gsem2, ssem0, ssem1, ssem2):
    cid = lax.axis_index("c")
    sid = lax.axis_index("s")
    wid = cid * NS + sid
    rows = (rows0, rows1, rows2)
    gsems = (gsem0, gsem1, gsem2)
    ssems = (ssem0, ssem1, ssem2)

    zero16 = jnp.zeros((16,), jnp.float32)

    def zb(i, c):
        for j in range(8):
            rows1[i, pl.ds(j * 16, 16)] = zero16
        return c

    lax.fori_loop(0, ZR, zb, 0)

    for k in range((NZB + NS - 1) // NS):
        b = k * NS + sid

        @pl.when(b < NZB)
        def _():
            base = pl.multiple_of(b * ZR, 8)
            pltpu.sync_copy(rows1, acc.at[pl.ds(base, ZR)])

    plsc.subcore_barrier()

    def issue_gather(g, buf):
        pltpu.async_copy(table_hbm.at[srcv.at[g]], rows[buf], gsems[buf])

    def wait_gather(g, buf):
        pltpu.make_async_copy(
            table_hbm.at[srcv.at[g]], rows[buf], gsems[buf]).wait()

    def issue_scatter(g, buf):
        pltpu.async_copy(rows[buf], acc.at[dstv.at[g]], ssems[buf], add=True)

    def wait_scatter(g, buf):
        pltpu.make_async_copy(
            rows[buf], acc.at[dstv.at[g]], ssems[buf]).wait()

    def scale(g, buf):
        def grp_body(grp, cc):
            vv = valv[g, pl.ds(grp * 16, 16)]
            rbase = grp * 16
            for r in range(16):
                v = vv[r]
                for j in range(8):
                    sl = pl.ds(j * 16, 16)
                    rows[buf][rbase + r, sl] = rows[buf][rbase + r, sl] * v
            return cc

        lax.fori_loop(0, CHUNK // 16, grp_body, 0)

    def super_block(sb, c):
        pltpu.sync_copy(src_hbm.at[wid, sb], srcv)
        pltpu.sync_copy(dst_hbm.at[wid, sb], dstv)
        pltpu.sync_copy(vals_hbm.at[wid, sb], valv)
        issue_gather(0, 0)

        issue_gather(1, 1)

        def do_chunk(g, buf, issue_ahead):
            wait_gather(g, buf)
            scale(g, buf)
            pltpu.sync_copy(rows[buf], acc.at[dstv.at[g]], add=True)
            if issue_ahead:
                issue_gather(g + 2, (buf + 2) % 3)

        def triple(i, cc):
            gb = i * 3
            do_chunk(gb, 0, True)
            do_chunk(gb + 1, 1, True)
            do_chunk(gb + 2, 2, True)
            return cc

        lax.fori_loop(0, (SBC - 4) // 3, triple, 0)
        do_chunk(SBC - 4, (SBC - 4) % 3, True)
        do_chunk(SBC - 3, (SBC - 3) % 3, True)
        do_chunk(SBC - 2, (SBC - 2) % 3, False)
        do_chunk(SBC - 1, (SBC - 1) % 3, False)
        return c

    lax.fori_loop(0, NSB, super_block, 0)
    plsc.subcore_barrier()

    for k in range((NZB + NS - 1) // NS):
        b = k * NS + sid

        @pl.when(b < NZB)
        def _():
            base = pl.multiple_of(b * ZR, 8)
            pltpu.sync_copy(acc.at[pl.ds(base, ZR)], out_hbm.at[cid, pl.ds(base, ZR)])


_spmm = pl.kernel(
    _spmm_body,
    out_type=jax.ShapeDtypeStruct((NC, N, D), jnp.float32),
    mesh=plsc.VectorSubcoreMesh(core_axis_name="c", subcore_axis_name="s"),
    scratch_types=[
        pltpu.VMEM_SHARED((N, D), jnp.float32),
        pltpu.VMEM((SBC, CHUNK), jnp.int32),
        pltpu.VMEM((SBC, CHUNK), jnp.int32),
        pltpu.VMEM((SBC, CHUNK), jnp.float32),
        pltpu.VMEM((CHUNK, D), jnp.float32),
        pltpu.VMEM((CHUNK, D), jnp.float32),
        pltpu.VMEM((CHUNK, D), jnp.float32),
        pltpu.SemaphoreType.DMA,
        pltpu.SemaphoreType.DMA,
        pltpu.SemaphoreType.DMA,
        pltpu.SemaphoreType.DMA,
        pltpu.SemaphoreType.DMA,
        pltpu.SemaphoreType.DMA,
    ],
)


def _tc1_body(lf_ref, f_ref, w1_ref, b1_ref, inter_ref, part1_ref):
    lf = lf_ref[0] + lf_ref[1]
    f = f_ref[...]
    inter_ref[...] = lf * f
    part1_ref[...] = lax.dot_general(
        lf + f, w1_ref[...], (((1,), (1,)), ((), ())),
        preferred_element_type=jnp.float32) + b1_ref[...]


BR = 2000

_tc1 = pl.pallas_call(
    _tc1_body,
    grid=(N // BR,),
    in_specs=[
        pl.BlockSpec((NC, BR, D), lambda i: (0, i, 0)),
        pl.BlockSpec((BR, D), lambda i: (i, 0)),
        pl.BlockSpec((D, D), lambda i: (0, 0)),
        pl.BlockSpec((1, D), lambda i: (0, 0)),
    ],
    out_specs=[
        pl.BlockSpec((BR, D), lambda i: (i, 0)),
        pl.BlockSpec((BR, D), lambda i: (i, 0)),
    ],
    out_shape=[
        jax.ShapeDtypeStruct((N, D), jnp.float32),
        jax.ShapeDtypeStruct((N, D), jnp.float32),
    ],
)


def _tc2_body(part1_ref, p_ref, w2_ref, b2_ref, out_ref):
    p = p_ref[0] + p_ref[1]
    out_ref[...] = part1_ref[...] + lax.dot_general(
        p, w2_ref[...], (((1,), (1,)), ((), ())),
        preferred_element_type=jnp.float32) + b2_ref[...]


_tc2 = pl.pallas_call(
    _tc2_body,
    grid=(N // BR,),
    in_specs=[
        pl.BlockSpec((BR, D), lambda i: (i, 0)),
        pl.BlockSpec((NC, BR, D), lambda i: (0, i, 0)),
        pl.BlockSpec((D, D), lambda i: (0, 0)),
        pl.BlockSpec((1, D), lambda i: (0, 0)),
    ],
    out_specs=pl.BlockSpec((BR, D), lambda i: (i, 0)),
    out_shape=jax.ShapeDtypeStruct((N, D), jnp.float32),
)


def kernel(laplacian_indices, laplacian_values, features, W1, b1, W2, b2):
    dst = laplacian_indices[0].reshape(NW, NSB, SBC, CHUNK)
    src = laplacian_indices[1].reshape(NW, NSB, SBC, CHUNK)
    vals = laplacian_values.reshape(NW, NSB, SBC, CHUNK)
    lf_parts = _spmm(src, dst, vals, features)
    inter, part1 = _tc1(lf_parts, features, W1, b1.reshape(1, D))
    p_parts = _spmm(src, dst, vals, inter)
    return _tc2(part1, p_parts, W2, b2.reshape(1, D))
